# baseline (speedup 1.0000x reference)
import functools

import jax
import jax.numpy as jnp
import numpy as np
from jax import lax
from jax.experimental import pallas as pl
from jax.experimental.pallas import tpu as pltpu
from jax.experimental.pallas import tpu_sc as plsc

B = 4096
D = 64
HIST = 50
K = 2
PK = 2 * D

_NC = 2
_NS = 16
_NW = _NC * _NS
_HPW = B * HIST // _NW
_CH = 128
_NCH = _HPW // _CH
_SPW = B // _NW


def _sc_gather_item_body(hist_ids, item_idx, emb_item,
                         out_hist, out_item,
                         ids_v, idx_v, par_v, rows_v, sel_v,
                         sidx_v, srows_v, sem):
    wid = lax.axis_index("s") * _NC + lax.axis_index("c")
    hbase = wid * _HPW
    iot = lax.broadcasted_iota(jnp.int32, (16,), 0)

    def chunk(j, carry):
        pltpu.sync_copy(hist_ids.at[pl.ds(hbase + j * _CH, _CH)], ids_v)
        for g in range(_CH // 16):
            id16 = ids_v[pl.ds(g * 16, 16)]
            ge = id16 >= _SPLIT
            idx_v[pl.ds(g * 16, 16)] = jnp.where(ge, id16 - _SPLIT, id16)
            par_v[pl.ds(g * 16, 16)] = jnp.where(ge, D, 0)
        pltpu.async_copy(emb_item.at[idx_v], rows_v, sem).wait()
        for g in range(_CH // 16):
            row16 = g * 16 + iot
            coff = par_v[pl.ds(g * 16, 16)]

            def cloop(c, carry2):
                vals = plsc.load_gather(rows_v, [row16, coff + c])
                plsc.store_scatter(sel_v, [row16, jnp.broadcast_to(c, (16,))],
                                   vals)
                return carry2

            lax.fori_loop(0, D, cloop, 0)
        pltpu.sync_copy(sel_v, out_hist.at[pl.ds(hbase + j * _CH, _CH)])
        return carry

    lax.fori_loop(0, _NCH, chunk, 0)

    sbase = wid * _SPW
    pltpu.sync_copy(item_idx.at[pl.ds(sbase, _SPW)], sidx_v)
    pltpu.async_copy(emb_item.at[sidx_v], srows_v, sem).wait()
    pltpu.sync_copy(srows_v, out_item.at[pl.ds(sbase, _SPW)])


def _sc_gather_user_body(user_idx, emb_user, out_user, sidx_v, srows_v, sem):
    wid = lax.axis_index("s") * _NC + lax.axis_index("c")
    sbase = wid * _SPW
    pltpu.sync_copy(user_idx.at[pl.ds(sbase, _SPW)], sidx_v)
    pltpu.async_copy(emb_user.at[sidx_v], srows_v, sem).wait()
    pltpu.sync_copy(srows_v, out_user.at[pl.ds(sbase, _SPW)])


@functools.lru_cache(maxsize=1)
def _make_sc_gathers():
    gi = pl.kernel(
        _sc_gather_item_body,
        out_type=[
            jax.ShapeDtypeStruct((B * HIST, D), jnp.float32),
            jax.ShapeDtypeStruct((B, PK), jnp.float32),
        ],
        mesh=plsc.VectorSubcoreMesh(core_axis_name="c", subcore_axis_name="s"),
        compiler_params=pltpu.CompilerParams(needs_layout_passes=False),
        scratch_types=[
            pltpu.VMEM((_CH,), jnp.int32),
            pltpu.VMEM((_CH,), jnp.int32),
            pltpu.VMEM((_CH,), jnp.int32),
            pltpu.VMEM((_CH, PK), jnp.float32),
            pltpu.VMEM((_CH, D), jnp.float32),
            pltpu.VMEM((_SPW,), jnp.int32),
            pltpu.VMEM((_SPW, PK), jnp.float32),
            pltpu.SemaphoreType.DMA,
        ],
    )
    gu = pl.kernel(
        _sc_gather_user_body,
        out_type=[jax.ShapeDtypeStruct((B, PK), jnp.float32)],
        mesh=plsc.VectorSubcoreMesh(core_axis_name="c", subcore_axis_name="s"),
        scratch_types=[
            pltpu.VMEM((_SPW,), jnp.int32),
            pltpu.VMEM((_SPW, PK), jnp.float32),
            pltpu.SemaphoreType.DMA,
        ],
    )
    return gi, gu


_VOCAB = 1000000
_SPLIT = 499968
_CVB = 7936
_CVG = 64
_PR = _CVG * _CVB


def _convert_body(xA, xB, out):
    eye = (lax.broadcasted_iota(jnp.int32, (D, D), 0)
           == lax.broadcasted_iota(jnp.int32, (D, D), 1)).astype(jnp.float32)
    tA = lax.dot_general(xA[...], eye, (((0,), (0,)), ((), ())),
                         preferred_element_type=jnp.float32)
    tB = lax.dot_general(xB[...], eye, (((0,), (0,)), ((), ())),
                         preferred_element_type=jnp.float32)
    out[...] = jnp.concatenate([tA, tB], axis=1)


_convert = pl.pallas_call(
    _convert_body,
    grid=(_CVG,),
    in_specs=[pl.BlockSpec((D, _CVB), lambda i: (0, i)),
              pl.BlockSpec((D, _CVB), lambda i: (0, i + 63))],
    out_specs=pl.BlockSpec((_CVB, PK), lambda i: (i, 0)),
    out_shape=jax.ShapeDtypeStruct((_PR, PK), jnp.float32),
    compiler_params=pltpu.CompilerParams(
        vmem_limit_bytes=64 * 2**20, fuse_transposed_lhs_in_matmul=True),
)


def _half_select(packed, ids):
    par = (ids[...] >= _SPLIT).astype(jnp.float32)
    left = packed[:, 0:D]
    right = packed[:, D:PK]
    return left + (right - left) * par


def _item_dnn_body(rows, ids, iW1, ib1, iW2, ib2, out):
    x = _half_select(rows[...], ids)
    h = jnp.maximum(
        jnp.dot(x, iW1[...], preferred_element_type=jnp.float32)
        + ib1[...], 0.0)
    out[...] = jnp.maximum(
        jnp.dot(h, iW2[...], preferred_element_type=jnp.float32)
        + ib2[...], 0.0)


_item_dnn = pl.pallas_call(
    _item_dnn_body,
    out_shape=jax.ShapeDtypeStruct((B, 32), jnp.float32),
)


_BB = 128


def _squash(x):
    s = jnp.sum(jnp.square(x), axis=-1, keepdims=True)
    return (s / (1.0 + s)) * x / jnp.sqrt(s + 1e-9)


def _main_body(hist, user, user_ids, gender_i, len_i, item_all,
               item_blk, embG, embL, Wcap, r2, uW1, ub1, uW2, ub2, out):
    u2 = jnp.dot(hist[...], Wcap[...], preferred_element_type=jnp.float32)
    u_hat = u2.reshape(_BB, HIST, D)
    seq = jnp.maximum(len_i[...].astype(jnp.float32), 1.0)
    mask = (lax.broadcasted_iota(jnp.int32, (_BB, HIST), 1).astype(jnp.float32)
            < seq).astype(jnp.float32)
    b0 = jnp.broadcast_to(r2[0:1, :], (_BB, HIST))
    b1 = jnp.broadcast_to(r2[1:2, :], (_BB, HIST))
    cap0 = cap1 = None
    for i in range(3):
        mx = jnp.maximum(b0, b1)
        e0 = jnp.exp(b0 - mx)
        e1 = jnp.exp(b1 - mx)
        den = e0 + e1
        w0 = e0 / den * mask
        w1 = e1 / den * mask
        Z0 = jnp.sum(w0[:, :, None] * u_hat, axis=1)
        Z1 = jnp.sum(w1[:, :, None] * u_hat, axis=1)
        cap0 = _squash(Z0)
        cap1 = _squash(Z1)
        if i < 2:
            b0 = b0 + jnp.sum(cap0[:, None, :] * u_hat, axis=2)
            b1 = b1 + jnp.sum(cap1[:, None, :] * u_hat, axis=2)
    user64 = _half_select(user[...], user_ids)
    gf = gender_i[...].astype(jnp.float32)
    gemb = embG[0:1, :] + (embG[1:2, :] - embG[0:1, :]) * gf
    lf = len_i[...]
    oh = (lax.broadcasted_iota(jnp.int32, (_BB, HIST + 1), 1)
          == lf).astype(jnp.float32)
    lemb = jnp.dot(oh, embL[...], preferred_element_type=jnp.float32)
    us = jnp.concatenate([user64, gemb, lemb], axis=1)
    ud0 = jnp.concatenate([us, cap0], axis=1)
    ud1 = jnp.concatenate([us, cap1], axis=1)
    h0 = jnp.maximum(jnp.dot(ud0, uW1[...], preferred_element_type=jnp.float32) + ub1[...], 0.0)
    h1 = jnp.maximum(jnp.dot(ud1, uW1[...], preferred_element_type=jnp.float32) + ub1[...], 0.0)
    o0 = jnp.maximum(jnp.dot(h0, uW2[...], preferred_element_type=jnp.float32) + ub2[...], 0.0)
    o1 = jnp.maximum(jnp.dot(h1, uW2[...], preferred_element_type=jnp.float32) + ub2[...], 0.0)
    it = item_blk[...]
    wt0 = jnp.sum(o0 * it, axis=1, keepdims=True)
    wt1 = jnp.sum(o1 * it, axis=1, keepdims=True)
    mw = jnp.maximum(wt0, wt1)
    a0 = jnp.exp(wt0 - mw)
    a1 = jnp.exp(wt1 - mw)
    uf = (o0 * a0 + o1 * a1) / (a0 + a1)
    logits = lax.dot_general(uf, item_all[...],
                             (((1,), (1,)), ((), ())),
                             preferred_element_type=jnp.float32)
    rowmax = jnp.max(logits, axis=1, keepdims=True)
    lse = jnp.log(jnp.sum(jnp.exp(logits - rowmax), axis=1,
                          keepdims=True)) + rowmax
    diag = jnp.sum(uf * it, axis=1, keepdims=True)
    out[...] = lse - diag


_main = pl.pallas_call(
    _main_body,
    grid=(B // _BB,),
    in_specs=[
        pl.BlockSpec((_BB * HIST, D), lambda i: (i, 0)),
        pl.BlockSpec((_BB, PK), lambda i: (i, 0)),
        pl.BlockSpec((_BB, 1), lambda i: (i, 0)),
        pl.BlockSpec((_BB, 1), lambda i: (i, 0)),
        pl.BlockSpec((_BB, 1), lambda i: (i, 0)),
        pl.BlockSpec((B, 32), lambda i: (0, 0)),
        pl.BlockSpec((_BB, 32), lambda i: (i, 0)),
        pl.BlockSpec((2, D), lambda i: (0, 0)),
        pl.BlockSpec((HIST + 1, D), lambda i: (0, 0)),
        pl.BlockSpec((D, D), lambda i: (0, 0)),
        pl.BlockSpec((K, HIST), lambda i: (0, 0)),
        pl.BlockSpec((4 * D, D), lambda i: (0, 0)),
        pl.BlockSpec((1, D), lambda i: (0, 0)),
        pl.BlockSpec((D, 32), lambda i: (0, 0)),
        pl.BlockSpec((1, 32), lambda i: (0, 0)),
    ],
    out_specs=pl.BlockSpec((_BB, 1), lambda i: (i, 0)),
    out_shape=jax.ShapeDtypeStruct((B, 1), jnp.float32),
    compiler_params=pltpu.CompilerParams(vmem_limit_bytes=64 * 2**20),
)


def kernel(user_id, gender, hist_len, item_id, hist_item_ids, labels,
           emb_user_id, emb_gender, emb_hist_len, emb_item_id,
           W_cap, routing_logits, uW1, ub1, uW2, ub2, iW1, ib1, iW2, ib2):
    emb_userT = emb_user_id.T
    emb_itemT = emb_item_id.T

    def _pidx(ids):
        return jnp.where(ids < _SPLIT, ids, ids - _SPLIT)

    gi, gu = _make_sc_gathers()
    emb_item128 = _convert(emb_itemT, emb_itemT)
    out_hist, out_item = gi(
        hist_item_ids.reshape(B * HIST), _pidx(item_id.reshape(B)),
        emb_item128)
    emb_user128 = _convert(emb_userT, emb_userT)
    (out_user,) = gu(_pidx(user_id.reshape(B)), emb_user128)
    item_dnn = _item_dnn(out_item, item_id, iW1, ib1.reshape(1, D),
                         iW2, ib2.reshape(1, 32))
    loss = _main(out_hist,
                 out_user, user_id, gender, hist_len,
                 item_dnn, item_dnn, emb_gender, emb_hist_len,
                 W_cap, routing_logits.reshape(K, HIST),
                 uW1, ub1.reshape(1, D), uW2, ub2.reshape(1, 32))
    return loss


# speedup vs baseline: 1.2335x; 1.2335x over previous
import functools

import jax
import jax.numpy as jnp
import numpy as np
from jax import lax
from jax.experimental import pallas as pl
from jax.experimental.pallas import tpu as pltpu
from jax.experimental.pallas import tpu_sc as plsc

B = 4096
D = 64
HIST = 50
K = 2
PK = 2 * D

_NC = 2
_NS = 16
_NW = _NC * _NS
_HPW = B * HIST // _NW
_CH = 128
_NCH = _HPW // _CH
_SPW = B // _NW


def _sc_gather_item_body(hist_idx, item_idx, emb_item,
                         out_hist, out_item,
                         idx_v, rows_v, sidx_v, srows_v, sem):
    wid = lax.axis_index("s") * _NC + lax.axis_index("c")
    hbase = wid * _HPW

    def chunk(j, carry):
        pltpu.sync_copy(hist_idx.at[pl.ds(hbase + j * _CH, _CH)], idx_v)
        pltpu.async_copy(emb_item.at[idx_v], rows_v, sem).wait()
        pltpu.sync_copy(rows_v, out_hist.at[pl.ds(hbase + j * _CH, _CH)])
        return carry

    lax.fori_loop(0, _NCH, chunk, 0)

    sbase = wid * _SPW
    pltpu.sync_copy(item_idx.at[pl.ds(sbase, _SPW)], sidx_v)
    pltpu.async_copy(emb_item.at[sidx_v], srows_v, sem).wait()
    pltpu.sync_copy(srows_v, out_item.at[pl.ds(sbase, _SPW)])


def _sc_gather_user_body(user_idx, emb_user, out_user, sidx_v, srows_v, sem):
    wid = lax.axis_index("s") * _NC + lax.axis_index("c")
    sbase = wid * _SPW
    pltpu.sync_copy(user_idx.at[pl.ds(sbase, _SPW)], sidx_v)
    pltpu.async_copy(emb_user.at[sidx_v], srows_v, sem).wait()
    pltpu.sync_copy(srows_v, out_user.at[pl.ds(sbase, _SPW)])


@functools.lru_cache(maxsize=1)
def _make_sc_gathers():
    gi = pl.kernel(
        _sc_gather_item_body,
        out_type=[
            jax.ShapeDtypeStruct((B * HIST, PK), jnp.float32),
            jax.ShapeDtypeStruct((B, PK), jnp.float32),
        ],
        mesh=plsc.VectorSubcoreMesh(core_axis_name="c", subcore_axis_name="s"),
        scratch_types=[
            pltpu.VMEM((_CH,), jnp.int32),
            pltpu.VMEM((_CH, PK), jnp.float32),
            pltpu.VMEM((_SPW,), jnp.int32),
            pltpu.VMEM((_SPW, PK), jnp.float32),
            pltpu.SemaphoreType.DMA,
        ],
    )
    gu = pl.kernel(
        _sc_gather_user_body,
        out_type=[jax.ShapeDtypeStruct((B, PK), jnp.float32)],
        mesh=plsc.VectorSubcoreMesh(core_axis_name="c", subcore_axis_name="s"),
        scratch_types=[
            pltpu.VMEM((_SPW,), jnp.int32),
            pltpu.VMEM((_SPW, PK), jnp.float32),
            pltpu.SemaphoreType.DMA,
        ],
    )
    return gi, gu


_VOCAB = 1000000
_SPLIT = 499968
_CVB = 7936
_CVG = 64
_PR = _CVG * _CVB


def _convert_body(xA, xB, out):
    eye = (lax.broadcasted_iota(jnp.int32, (D, D), 0)
           == lax.broadcasted_iota(jnp.int32, (D, D), 1)).astype(jnp.float32)
    tA = lax.dot_general(xA[...], eye, (((0,), (0,)), ((), ())),
                         preferred_element_type=jnp.float32)
    tB = lax.dot_general(xB[...], eye, (((0,), (0,)), ((), ())),
                         preferred_element_type=jnp.float32)
    out[...] = jnp.concatenate([tA, tB], axis=1)


_convert = pl.pallas_call(
    _convert_body,
    grid=(_CVG,),
    in_specs=[pl.BlockSpec((D, _CVB), lambda i: (0, i)),
              pl.BlockSpec((D, _CVB), lambda i: (0, i + 63))],
    out_specs=pl.BlockSpec((_CVB, PK), lambda i: (i, 0)),
    out_shape=jax.ShapeDtypeStruct((_PR, PK), jnp.float32),
    compiler_params=pltpu.CompilerParams(
        vmem_limit_bytes=64 * 2**20, fuse_transposed_lhs_in_matmul=True),
)


def _half_select(packed, ids):
    par = (ids[...] >= _SPLIT).astype(jnp.float32)
    left = packed[:, 0:D]
    right = packed[:, D:PK]
    return left + (right - left) * par


def _item_dnn_body(rows, ids, iW1, ib1, iW2, ib2, out):
    x = _half_select(rows[...], ids)
    h = jnp.maximum(
        jnp.dot(x, iW1[...], preferred_element_type=jnp.float32)
        + ib1[...], 0.0)
    out[...] = jnp.maximum(
        jnp.dot(h, iW2[...], preferred_element_type=jnp.float32)
        + ib2[...], 0.0)


_item_dnn = pl.pallas_call(
    _item_dnn_body,
    out_shape=jax.ShapeDtypeStruct((B, 32), jnp.float32),
)


_BB = 128


def _squash(x):
    s = jnp.sum(jnp.square(x), axis=-1, keepdims=True)
    return (s / (1.0 + s)) * x / jnp.sqrt(s + 1e-9)


def _main_body(hist, hist_ids, user, user_ids, gender_i, len_i, item_all,
               item_blk, embG, embL, Wcap, r2, uW1, ub1, uW2, ub2, out):
    hist64 = _half_select(hist[...], hist_ids)
    u2 = jnp.dot(hist64, Wcap[...], preferred_element_type=jnp.float32)
    u_hat = u2.reshape(_BB, HIST, D)
    seq = jnp.maximum(len_i[...].astype(jnp.float32), 1.0)
    mask = (lax.broadcasted_iota(jnp.int32, (_BB, HIST), 1).astype(jnp.float32)
            < seq).astype(jnp.float32)
    b0 = jnp.broadcast_to(r2[0:1, :], (_BB, HIST))
    b1 = jnp.broadcast_to(r2[1:2, :], (_BB, HIST))
    cap0 = cap1 = None
    for i in range(3):
        mx = jnp.maximum(b0, b1)
        e0 = jnp.exp(b0 - mx)
        e1 = jnp.exp(b1 - mx)
        den = e0 + e1
        w0 = e0 / den * mask
        w1 = e1 / den * mask
        Z0 = jnp.sum(w0[:, :, None] * u_hat, axis=1)
        Z1 = jnp.sum(w1[:, :, None] * u_hat, axis=1)
        cap0 = _squash(Z0)
        cap1 = _squash(Z1)
        if i < 2:
            b0 = b0 + jnp.sum(cap0[:, None, :] * u_hat, axis=2)
            b1 = b1 + jnp.sum(cap1[:, None, :] * u_hat, axis=2)
    user64 = _half_select(user[...], user_ids)
    gf = gender_i[...].astype(jnp.float32)
    gemb = embG[0:1, :] + (embG[1:2, :] - embG[0:1, :]) * gf
    lf = len_i[...]
    oh = (lax.broadcasted_iota(jnp.int32, (_BB, HIST + 1), 1)
          == lf).astype(jnp.float32)
    lemb = jnp.dot(oh, embL[...], preferred_element_type=jnp.float32)
    us = jnp.concatenate([user64, gemb, lemb], axis=1)
    ud0 = jnp.concatenate([us, cap0], axis=1)
    ud1 = jnp.concatenate([us, cap1], axis=1)
    h0 = jnp.maximum(jnp.dot(ud0, uW1[...], preferred_element_type=jnp.float32) + ub1[...], 0.0)
    h1 = jnp.maximum(jnp.dot(ud1, uW1[...], preferred_element_type=jnp.float32) + ub1[...], 0.0)
    o0 = jnp.maximum(jnp.dot(h0, uW2[...], preferred_element_type=jnp.float32) + ub2[...], 0.0)
    o1 = jnp.maximum(jnp.dot(h1, uW2[...], preferred_element_type=jnp.float32) + ub2[...], 0.0)
    it = item_blk[...]
    wt0 = jnp.sum(o0 * it, axis=1, keepdims=True)
    wt1 = jnp.sum(o1 * it, axis=1, keepdims=True)
    mw = jnp.maximum(wt0, wt1)
    a0 = jnp.exp(wt0 - mw)
    a1 = jnp.exp(wt1 - mw)
    uf = (o0 * a0 + o1 * a1) / (a0 + a1)
    logits = lax.dot_general(uf, item_all[...],
                             (((1,), (1,)), ((), ())),
                             preferred_element_type=jnp.float32)
    rowmax = jnp.max(logits, axis=1, keepdims=True)
    lse = jnp.log(jnp.sum(jnp.exp(logits - rowmax), axis=1,
                          keepdims=True)) + rowmax
    diag = jnp.sum(uf * it, axis=1, keepdims=True)
    out[...] = lse - diag


_main = pl.pallas_call(
    _main_body,
    grid=(B // _BB,),
    in_specs=[
        pl.BlockSpec((_BB * HIST, PK), lambda i: (i, 0)),
        pl.BlockSpec((_BB * HIST, 1), lambda i: (i, 0)),
        pl.BlockSpec((_BB, PK), lambda i: (i, 0)),
        pl.BlockSpec((_BB, 1), lambda i: (i, 0)),
        pl.BlockSpec((_BB, 1), lambda i: (i, 0)),
        pl.BlockSpec((_BB, 1), lambda i: (i, 0)),
        pl.BlockSpec((B, 32), lambda i: (0, 0)),
        pl.BlockSpec((_BB, 32), lambda i: (i, 0)),
        pl.BlockSpec((2, D), lambda i: (0, 0)),
        pl.BlockSpec((HIST + 1, D), lambda i: (0, 0)),
        pl.BlockSpec((D, D), lambda i: (0, 0)),
        pl.BlockSpec((K, HIST), lambda i: (0, 0)),
        pl.BlockSpec((4 * D, D), lambda i: (0, 0)),
        pl.BlockSpec((1, D), lambda i: (0, 0)),
        pl.BlockSpec((D, 32), lambda i: (0, 0)),
        pl.BlockSpec((1, 32), lambda i: (0, 0)),
    ],
    out_specs=pl.BlockSpec((_BB, 1), lambda i: (i, 0)),
    out_shape=jax.ShapeDtypeStruct((B, 1), jnp.float32),
    compiler_params=pltpu.CompilerParams(vmem_limit_bytes=64 * 2**20),
)


def kernel(user_id, gender, hist_len, item_id, hist_item_ids, labels,
           emb_user_id, emb_gender, emb_hist_len, emb_item_id,
           W_cap, routing_logits, uW1, ub1, uW2, ub2, iW1, ib1, iW2, ib2):
    emb_userT = emb_user_id.T
    emb_itemT = emb_item_id.T

    def _pidx(ids):
        return jnp.where(ids < _SPLIT, ids, ids - _SPLIT)

    gi, gu = _make_sc_gathers()
    emb_item128 = _convert(emb_itemT, emb_itemT)
    out_hist, out_item = gi(
        _pidx(hist_item_ids.reshape(B * HIST)), _pidx(item_id.reshape(B)),
        emb_item128)
    emb_user128 = _convert(emb_userT, emb_userT)
    (out_user,) = gu(_pidx(user_id.reshape(B)), emb_user128)
    item_dnn = _item_dnn(out_item, item_id, iW1, ib1.reshape(1, D),
                         iW2, ib2.reshape(1, 32))
    loss = _main(out_hist, hist_item_ids.reshape(B * HIST, 1),
                 out_user, user_id, gender, hist_len,
                 item_dnn, item_dnn, emb_gender, emb_hist_len,
                 W_cap, routing_logits.reshape(K, HIST),
                 uW1, ub1.reshape(1, D), uW2, ub2.reshape(1, 32))
    return loss
